# baseline (speedup 1.0000x reference)
import jax
import jax.numpy as jnp
from jax import lax
from jax.experimental import pallas as pl
from jax.experimental.pallas import tpu as pltpu
from jax.experimental.pallas import tpu_sc as plsc

B = 16384
L = 50
D = 64
DP = 128
N = B * L

NC = 2
NS = 16
NW = NC * NS
LANES = 16

PAIRS_PER_W = N // NW
CHUNK = 128
CB = 10
BLKSZ = CB * CHUNK
NBLK = PAIRS_PER_W // BLKSZ
PU = 4

TR_BLK = 2048


def _transpose_body(x_ref, o_ref):
  eye = (lax.broadcasted_iota(jnp.int32, (D, D), 0)
         == lax.broadcasted_iota(jnp.int32, (D, D), 1)).astype(jnp.float32)
  o_ref[:, 0:D] = lax.dot_general(
      x_ref[...], eye, (((0,), (0,)), ((), ())),
      preferred_element_type=jnp.float32)


def _to_row_major(table_t):
  n = table_t.shape[1]
  return pl.pallas_call(
      _transpose_body,
      grid=((n + TR_BLK - 1) // TR_BLK,),
      in_specs=[pl.BlockSpec((D, TR_BLK), lambda i: (0, i))],
      out_specs=pl.BlockSpec((TR_BLK, DP), lambda i: (i, 0)),
      out_shape=jax.ShapeDtypeStruct((n, DP), jnp.float32),
      compiler_params=pltpu.CompilerParams(
          fuse_transposed_lhs_in_matmul=True),
  )(table_t)


def _body(widx_hbm, cidx_hbm, table_hbm, fcw_hbm, fcb_hbm, out_hbm,
          idx_w, idx_c, rows_w, rows_c, out_v, fcw_v, fcb_v,
          ids0, ids1, rs0, rs1, os0, os1):
  wid = lax.axis_index("s") * NC + lax.axis_index("c")
  idrow0 = wid * (PAIRS_PER_W // CHUNK)
  base0 = wid * PAIRS_PER_W
  idsem = (ids0, ids1)
  rowsem = (rs0, rs1)
  outsem = (os0, os1)

  pltpu.sync_copy(fcw_hbm, fcw_v)
  pltpu.sync_copy(fcb_hbm, fcb_v)
  f = [fcw_v[pl.ds(16 * k, 16)] for k in range(D // 16)]
  fb = fcb_v[...]
  lane15 = lax.iota(jnp.int32, 16) == 15

  def fire_ids(blk, b):
    rb = idrow0 + blk * CB
    pltpu.async_copy(widx_hbm.at[pl.ds(rb, CB)], idx_w.at[b], idsem[b])
    pltpu.async_copy(cidx_hbm.at[pl.ds(rb, CB)], idx_c.at[b], idsem[b])

  def drain_ids(b):
    pltpu.make_async_copy(widx_hbm.at[pl.ds(0, CB)], idx_w.at[b],
                          idsem[b]).wait()
    pltpu.make_async_copy(cidx_hbm.at[pl.ds(0, CB)], idx_c.at[b],
                          idsem[b]).wait()

  def fire_rows(iw_row, ic_row, p):
    pltpu.async_copy(table_hbm.at[iw_row], rows_w.at[p], rowsem[p])
    pltpu.async_copy(table_hbm.at[ic_row], rows_c.at[p], rowsem[p])

  def drain_rows(p):
    pltpu.make_async_copy(table_hbm.at[pl.ds(0, CHUNK)], rows_w.at[p],
                          rowsem[p]).wait()
    pltpu.make_async_copy(table_hbm.at[pl.ds(0, CHUNK)], rows_c.at[p],
                          rowsem[p]).wait()

  def compute_chunk(j, bb, p):
    off = j * CHUNK

    @plsc.parallel_loop(0, CHUNK, 1, unroll=PU)
    def _(pi):
      acc = (rows_w[p, pi, pl.ds(0, 16)]
             * rows_c[p, pi, pl.ds(0, 16)]) * f[0]
      for k in range(1, D // 16):
        acc += (rows_w[p, pi, pl.ds(16 * k, 16)]
                * rows_c[p, pi, pl.ds(16 * k, 16)]) * f[k]
      cs = jnp.cumsum(acc)
      plsc.store_compressed(out_v.at[bb].at[pl.ds(off + pi, 16)], cs,
                            mask=lane15)

  fire_ids(0, 0)
  drain_ids(0)
  fire_rows(idx_w.at[0].at[0], idx_c.at[0].at[0], 0)

  def block_loop(t, _):
    for sb in (0, 1):
      blk = 2 * t + sb
      bb = sb

      @pl.when(blk + 1 < NBLK)
      def _():
        fire_ids(blk + 1, 1 - bb)

      @pl.when(blk >= 2)
      def _():
        pltpu.make_async_copy(out_v.at[bb].at[pl.ds(0, BLKSZ)],
                              out_hbm.at[pl.ds(0, BLKSZ)],
                              outsem[bb]).wait()

      for j in range(CB):
        p = j % 2
        if j < CB - 1:
          fire_rows(idx_w.at[bb].at[j + 1], idx_c.at[bb].at[j + 1], 1 - p)
        else:
          @pl.when(blk + 1 < NBLK)
          def _():
            drain_ids(1 - bb)
            fire_rows(idx_w.at[1 - bb].at[0], idx_c.at[1 - bb].at[0], 1 - p)
        drain_rows(p)
        compute_chunk(j, bb, p)

      def sig_block(s, _):
        v = out_v[bb, pl.ds(s * 16, 16)] + fb
        out_v[bb, pl.ds(s * 16, 16)] = 1.0 / (1.0 + jnp.exp(-v))
        return 0

      lax.fori_loop(0, BLKSZ // 16, sig_block, 0)

      pltpu.async_copy(out_v.at[bb].at[pl.ds(0, BLKSZ)],
                       out_hbm.at[pl.ds(base0 + blk * BLKSZ, BLKSZ)],
                       outsem[bb])
    return 0

  lax.fori_loop(0, NBLK // 2, block_loop, 0)
  for bb in (0, 1):
    pltpu.make_async_copy(out_v.at[bb].at[pl.ds(0, BLKSZ)],
                          out_hbm.at[pl.ds(0, BLKSZ)], outsem[bb]).wait()


@jax.jit
def _run(widx, cidx, table, fcw, fcb16):
  mesh = plsc.VectorSubcoreMesh(
      core_axis_name="c", subcore_axis_name="s",
      num_cores=NC, num_subcores=NS)
  return pl.kernel(
      _body,
      out_type=jax.ShapeDtypeStruct((N,), jnp.float32),
      mesh=mesh,
      compiler_params=pltpu.CompilerParams(
          needs_layout_passes=False, use_tc_tiling_on_sc=False),
      scratch_types=[
          pltpu.VMEM((2, CB, CHUNK), jnp.int32),
          pltpu.VMEM((2, CB, CHUNK), jnp.int32),
          pltpu.VMEM((2, CHUNK, D), jnp.float32),
          pltpu.VMEM((2, CHUNK, D), jnp.float32),
          pltpu.VMEM((2, BLKSZ + LANES), jnp.float32),
          pltpu.VMEM((D,), jnp.float32),
          pltpu.VMEM((LANES,), jnp.float32),
          pltpu.SemaphoreType.DMA,
          pltpu.SemaphoreType.DMA,
          pltpu.SemaphoreType.DMA,
          pltpu.SemaphoreType.DMA,
          pltpu.SemaphoreType.DMA,
          pltpu.SemaphoreType.DMA,
      ],
  )(widx, cidx, table, fcw, fcb16)


def kernel(word_ids, context_ids, table, fc_w, fc_b):
  widx = (word_ids.astype(jnp.int32) * 2).reshape(N // CHUNK, CHUNK)
  cidx = (context_ids.astype(jnp.int32) * 2).reshape(N // CHUNK, CHUNK)
  fcw = fc_w.reshape(D)
  fcb16 = jnp.broadcast_to(fc_b.reshape(1), (LANES,))
  table_rm = _to_row_major(jnp.swapaxes(table, 0, 1))
  out = _run(widx, cidx, table_rm.reshape(2 * 1000000, D), fcw, fcb16)
  return out.reshape(B, L, 1)


# speedup vs baseline: 1.3806x; 1.3806x over previous
import jax
import jax.numpy as jnp
from jax import lax
from jax.experimental import pallas as pl
from jax.experimental.pallas import tpu as pltpu
from jax.experimental.pallas import tpu_sc as plsc

B = 16384
L = 50
D = 64
DP = 128
N = B * L

NC = 2
NS = 16
NW = NC * NS
LANES = 16

PAIRS_PER_W = N // NW
CHUNK = 128
CB = 10
BLKSZ = CB * CHUNK
NBLK = PAIRS_PER_W // BLKSZ
PU = 4

TR_BLK = 8192


def _transpose_body(x_ref, o_ref):
  o_ref[:, 0:D] = x_ref[...].T


def _to_row_major(table_t):
  n = table_t.shape[1]
  return pl.pallas_call(
      _transpose_body,
      grid=((n + TR_BLK - 1) // TR_BLK,),
      in_specs=[pl.BlockSpec((D, TR_BLK), lambda i: (0, i))],
      out_specs=pl.BlockSpec((TR_BLK, DP), lambda i: (i, 0)),
      out_shape=jax.ShapeDtypeStruct((n, DP), jnp.float32),
  )(table_t)


def _body(widx_hbm, cidx_hbm, table_hbm, fcw_hbm, fcb_hbm, out_hbm,
          idx_w, idx_c, rows_w, rows_c, out_v, fcw_v, fcb_v,
          ids0, ids1, rs0, rs1, os0, os1):
  wid = lax.axis_index("s") * NC + lax.axis_index("c")
  idrow0 = wid * (PAIRS_PER_W // CHUNK)
  base0 = wid * PAIRS_PER_W
  idsem = (ids0, ids1)
  rowsem = (rs0, rs1)
  outsem = (os0, os1)

  pltpu.sync_copy(fcw_hbm, fcw_v)
  pltpu.sync_copy(fcb_hbm, fcb_v)
  f = [fcw_v[pl.ds(16 * k, 16)] for k in range(D // 16)]
  fb = fcb_v[...]
  lane15 = lax.iota(jnp.int32, 16) == 15

  def fire_ids(blk, b):
    rb = idrow0 + blk * CB
    pltpu.async_copy(widx_hbm.at[pl.ds(rb, CB)], idx_w.at[b], idsem[b])
    pltpu.async_copy(cidx_hbm.at[pl.ds(rb, CB)], idx_c.at[b], idsem[b])

  def drain_ids(b):
    pltpu.make_async_copy(widx_hbm.at[pl.ds(0, CB)], idx_w.at[b],
                          idsem[b]).wait()
    pltpu.make_async_copy(cidx_hbm.at[pl.ds(0, CB)], idx_c.at[b],
                          idsem[b]).wait()

  def fire_rows(iw_row, ic_row, p):
    pltpu.async_copy(table_hbm.at[iw_row], rows_w.at[p], rowsem[p])
    pltpu.async_copy(table_hbm.at[ic_row], rows_c.at[p], rowsem[p])

  def drain_rows(p):
    pltpu.make_async_copy(table_hbm.at[pl.ds(0, CHUNK)], rows_w.at[p],
                          rowsem[p]).wait()
    pltpu.make_async_copy(table_hbm.at[pl.ds(0, CHUNK)], rows_c.at[p],
                          rowsem[p]).wait()

  def compute_chunk(j, bb, p):
    off = j * CHUNK

    @plsc.parallel_loop(0, CHUNK, 1, unroll=PU)
    def _(pi):
      acc = (rows_w[p, pi, pl.ds(0, 16)]
             * rows_c[p, pi, pl.ds(0, 16)]) * f[0]
      for k in range(1, D // 16):
        acc += (rows_w[p, pi, pl.ds(16 * k, 16)]
                * rows_c[p, pi, pl.ds(16 * k, 16)]) * f[k]
      cs = jnp.cumsum(acc)
      plsc.store_compressed(out_v.at[bb].at[pl.ds(off + pi, 16)], cs,
                            mask=lane15)

  fire_ids(0, 0)
  drain_ids(0)
  fire_rows(idx_w.at[0].at[0], idx_c.at[0].at[0], 0)

  def block_loop(t, _):
    for sb in (0, 1):
      blk = 2 * t + sb
      bb = sb

      @pl.when(blk + 1 < NBLK)
      def _():
        fire_ids(blk + 1, 1 - bb)

      @pl.when(blk >= 2)
      def _():
        pltpu.make_async_copy(out_v.at[bb].at[pl.ds(0, BLKSZ)],
                              out_hbm.at[pl.ds(0, BLKSZ)],
                              outsem[bb]).wait()

      for j in range(CB):
        p = j % 2
        if j < CB - 1:
          fire_rows(idx_w.at[bb].at[j + 1], idx_c.at[bb].at[j + 1], 1 - p)
        else:
          @pl.when(blk + 1 < NBLK)
          def _():
            drain_ids(1 - bb)
            fire_rows(idx_w.at[1 - bb].at[0], idx_c.at[1 - bb].at[0], 1 - p)
        drain_rows(p)
        compute_chunk(j, bb, p)

      def sig_block(s, _):
        v = out_v[bb, pl.ds(s * 16, 16)] + fb
        out_v[bb, pl.ds(s * 16, 16)] = 1.0 / (1.0 + jnp.exp(-v))
        return 0

      lax.fori_loop(0, BLKSZ // 16, sig_block, 0)

      pltpu.async_copy(out_v.at[bb].at[pl.ds(0, BLKSZ)],
                       out_hbm.at[pl.ds(base0 + blk * BLKSZ, BLKSZ)],
                       outsem[bb])
    return 0

  lax.fori_loop(0, NBLK // 2, block_loop, 0)
  for bb in (0, 1):
    pltpu.make_async_copy(out_v.at[bb].at[pl.ds(0, BLKSZ)],
                          out_hbm.at[pl.ds(0, BLKSZ)], outsem[bb]).wait()


@jax.jit
def _run(widx, cidx, table, fcw, fcb16):
  mesh = plsc.VectorSubcoreMesh(
      core_axis_name="c", subcore_axis_name="s",
      num_cores=NC, num_subcores=NS)
  return pl.kernel(
      _body,
      out_type=jax.ShapeDtypeStruct((N,), jnp.float32),
      mesh=mesh,
      compiler_params=pltpu.CompilerParams(
          needs_layout_passes=False, use_tc_tiling_on_sc=False),
      scratch_types=[
          pltpu.VMEM((2, CB, CHUNK), jnp.int32),
          pltpu.VMEM((2, CB, CHUNK), jnp.int32),
          pltpu.VMEM((2, CHUNK, D), jnp.float32),
          pltpu.VMEM((2, CHUNK, D), jnp.float32),
          pltpu.VMEM((2, BLKSZ + LANES), jnp.float32),
          pltpu.VMEM((D,), jnp.float32),
          pltpu.VMEM((LANES,), jnp.float32),
          pltpu.SemaphoreType.DMA,
          pltpu.SemaphoreType.DMA,
          pltpu.SemaphoreType.DMA,
          pltpu.SemaphoreType.DMA,
          pltpu.SemaphoreType.DMA,
          pltpu.SemaphoreType.DMA,
      ],
  )(widx, cidx, table, fcw, fcb16)


def kernel(word_ids, context_ids, table, fc_w, fc_b):
  widx = (word_ids.astype(jnp.int32) * 2).reshape(N // CHUNK, CHUNK)
  cidx = (context_ids.astype(jnp.int32) * 2).reshape(N // CHUNK, CHUNK)
  fcw = fc_w.reshape(D)
  fcb16 = jnp.broadcast_to(fc_b.reshape(1), (LANES,))
  table_rm = _to_row_major(jnp.swapaxes(table, 0, 1))
  out = _run(widx, cidx, table_rm.reshape(2 * 1000000, D), fcw, fcb16)
  return out.reshape(B, L, 1)


# speedup vs baseline: 1.4349x; 1.0394x over previous
import jax
import jax.numpy as jnp
from jax import lax
from jax.experimental import pallas as pl
from jax.experimental.pallas import tpu as pltpu
from jax.experimental.pallas import tpu_sc as plsc

B = 16384
L = 50
D = 64
DP = 128
N = B * L

NC = 2
NS = 16
NW = NC * NS
LANES = 16

PAIRS_PER_W = N // NW
CHUNK = 128
CB = 10
BLKSZ = CB * CHUNK
NBLK = PAIRS_PER_W // BLKSZ
PU = 4

TR_BLK = 16384


def _transpose_body(x_ref, o_ref):
  o_ref[:, 0:D] = x_ref[...].T


def _to_row_major(table_t):
  n = table_t.shape[1]
  return pl.pallas_call(
      _transpose_body,
      grid=((n + TR_BLK - 1) // TR_BLK,),
      in_specs=[pl.BlockSpec((D, TR_BLK), lambda i: (0, i))],
      out_specs=pl.BlockSpec((TR_BLK, DP), lambda i: (i, 0)),
      out_shape=jax.ShapeDtypeStruct((n, DP), jnp.float32),
  )(table_t)


def _body(widx_hbm, cidx_hbm, table_hbm, fcw_hbm, fcb_hbm, out_hbm,
          idx_w, idx_c, rows_w, rows_c, out_v, fcw_v, fcb_v,
          ids0, ids1, rs0, rs1, os0, os1):
  wid = lax.axis_index("s") * NC + lax.axis_index("c")
  idrow0 = wid * (PAIRS_PER_W // CHUNK)
  base0 = wid * PAIRS_PER_W
  idsem = (ids0, ids1)
  rowsem = (rs0, rs1)
  outsem = (os0, os1)

  pltpu.sync_copy(fcw_hbm, fcw_v)
  pltpu.sync_copy(fcb_hbm, fcb_v)
  f = [fcw_v[pl.ds(16 * k, 16)] for k in range(D // 16)]
  fb = fcb_v[...]
  lane15 = lax.iota(jnp.int32, 16) == 15

  def fire_ids(blk, b):
    rb = idrow0 + blk * CB
    pltpu.async_copy(widx_hbm.at[pl.ds(rb, CB)], idx_w.at[b], idsem[b])
    pltpu.async_copy(cidx_hbm.at[pl.ds(rb, CB)], idx_c.at[b], idsem[b])

  def drain_ids(b):
    pltpu.make_async_copy(widx_hbm.at[pl.ds(0, CB)], idx_w.at[b],
                          idsem[b]).wait()
    pltpu.make_async_copy(cidx_hbm.at[pl.ds(0, CB)], idx_c.at[b],
                          idsem[b]).wait()

  def fire_rows(iw_row, ic_row, p):
    pltpu.async_copy(table_hbm.at[iw_row], rows_w.at[p], rowsem[p])
    pltpu.async_copy(table_hbm.at[ic_row], rows_c.at[p], rowsem[p])

  def drain_rows(p):
    pltpu.make_async_copy(table_hbm.at[pl.ds(0, CHUNK)], rows_w.at[p],
                          rowsem[p]).wait()
    pltpu.make_async_copy(table_hbm.at[pl.ds(0, CHUNK)], rows_c.at[p],
                          rowsem[p]).wait()

  def compute_chunk(j, bb, p):
    off = j * CHUNK

    @plsc.parallel_loop(0, CHUNK, 1, unroll=PU)
    def _(pi):
      acc = (rows_w[p, pi, pl.ds(0, 16)]
             * rows_c[p, pi, pl.ds(0, 16)]) * f[0]
      for k in range(1, D // 16):
        acc += (rows_w[p, pi, pl.ds(16 * k, 16)]
                * rows_c[p, pi, pl.ds(16 * k, 16)]) * f[k]
      cs = jnp.cumsum(acc)
      plsc.store_compressed(out_v.at[bb].at[pl.ds(off + pi, 16)], cs,
                            mask=lane15)

  fire_ids(0, 0)
  drain_ids(0)
  fire_rows(idx_w.at[0].at[0], idx_c.at[0].at[0], 0)

  def block_loop(t, _):
    for sb in (0, 1):
      blk = 2 * t + sb
      bb = sb

      @pl.when(blk + 1 < NBLK)
      def _():
        fire_ids(blk + 1, 1 - bb)

      @pl.when(blk >= 2)
      def _():
        pltpu.make_async_copy(out_v.at[bb].at[pl.ds(0, BLKSZ)],
                              out_hbm.at[pl.ds(0, BLKSZ)],
                              outsem[bb]).wait()

      for j in range(CB):
        p = j % 2
        if j < CB - 1:
          fire_rows(idx_w.at[bb].at[j + 1], idx_c.at[bb].at[j + 1], 1 - p)
        else:
          @pl.when(blk + 1 < NBLK)
          def _():
            drain_ids(1 - bb)
            fire_rows(idx_w.at[1 - bb].at[0], idx_c.at[1 - bb].at[0], 1 - p)
        drain_rows(p)
        compute_chunk(j, bb, p)

      def sig_block(s, _):
        v = out_v[bb, pl.ds(s * 16, 16)] + fb
        out_v[bb, pl.ds(s * 16, 16)] = 1.0 / (1.0 + jnp.exp(-v))
        return 0

      lax.fori_loop(0, BLKSZ // 16, sig_block, 0)

      pltpu.async_copy(out_v.at[bb].at[pl.ds(0, BLKSZ)],
                       out_hbm.at[pl.ds(base0 + blk * BLKSZ, BLKSZ)],
                       outsem[bb])
    return 0

  lax.fori_loop(0, NBLK // 2, block_loop, 0)
  for bb in (0, 1):
    pltpu.make_async_copy(out_v.at[bb].at[pl.ds(0, BLKSZ)],
                          out_hbm.at[pl.ds(0, BLKSZ)], outsem[bb]).wait()


@jax.jit
def _run(widx, cidx, table, fcw, fcb16):
  mesh = plsc.VectorSubcoreMesh(
      core_axis_name="c", subcore_axis_name="s",
      num_cores=NC, num_subcores=NS)
  return pl.kernel(
      _body,
      out_type=jax.ShapeDtypeStruct((N,), jnp.float32),
      mesh=mesh,
      compiler_params=pltpu.CompilerParams(
          needs_layout_passes=False, use_tc_tiling_on_sc=False),
      scratch_types=[
          pltpu.VMEM((2, CB, CHUNK), jnp.int32),
          pltpu.VMEM((2, CB, CHUNK), jnp.int32),
          pltpu.VMEM((2, CHUNK, D), jnp.float32),
          pltpu.VMEM((2, CHUNK, D), jnp.float32),
          pltpu.VMEM((2, BLKSZ + LANES), jnp.float32),
          pltpu.VMEM((D,), jnp.float32),
          pltpu.VMEM((LANES,), jnp.float32),
          pltpu.SemaphoreType.DMA,
          pltpu.SemaphoreType.DMA,
          pltpu.SemaphoreType.DMA,
          pltpu.SemaphoreType.DMA,
          pltpu.SemaphoreType.DMA,
          pltpu.SemaphoreType.DMA,
      ],
  )(widx, cidx, table, fcw, fcb16)


def kernel(word_ids, context_ids, table, fc_w, fc_b):
  widx = (word_ids.astype(jnp.int32) * 2).reshape(N // CHUNK, CHUNK)
  cidx = (context_ids.astype(jnp.int32) * 2).reshape(N // CHUNK, CHUNK)
  fcw = fc_w.reshape(D)
  fcb16 = jnp.broadcast_to(fc_b.reshape(1), (LANES,))
  table_rm = _to_row_major(jnp.swapaxes(table, 0, 1))
  out = _run(widx, cidx, table_rm.reshape(2 * 1000000, D), fcw, fcb16)
  return out.reshape(B, L, 1)


# speedup vs baseline: 1.4480x; 1.0091x over previous
import jax
import jax.numpy as jnp
from jax import lax
from jax.experimental import pallas as pl
from jax.experimental.pallas import tpu as pltpu
from jax.experimental.pallas import tpu_sc as plsc

B = 16384
L = 50
D = 64
DP = 128
N = B * L

NC = 2
NS = 16
NW = NC * NS
LANES = 16

PAIRS_PER_W = N // NW
CHUNK = 128
CB = 10
BLKSZ = CB * CHUNK
NBLK = PAIRS_PER_W // BLKSZ
PU = 4

TR_BLK = 32768


def _transpose_body(x_ref, o_ref):
  o_ref[:, 0:D] = x_ref[...].T


def _to_row_major(table_t):
  n = table_t.shape[1]
  return pl.pallas_call(
      _transpose_body,
      grid=((n + TR_BLK - 1) // TR_BLK,),
      in_specs=[pl.BlockSpec((D, TR_BLK), lambda i: (0, i))],
      out_specs=pl.BlockSpec((TR_BLK, DP), lambda i: (i, 0)),
      out_shape=jax.ShapeDtypeStruct((n, DP), jnp.float32),
  )(table_t)


def _body(widx_hbm, cidx_hbm, table_hbm, fcw_hbm, fcb_hbm, out_hbm,
          idx_w, idx_c, rows_w, rows_c, out_v, fcw_v, fcb_v,
          ids0, ids1, rs0, rs1, os0, os1):
  wid = lax.axis_index("s") * NC + lax.axis_index("c")
  idrow0 = wid * (PAIRS_PER_W // CHUNK)
  base0 = wid * PAIRS_PER_W
  idsem = (ids0, ids1)
  rowsem = (rs0, rs1)
  outsem = (os0, os1)

  pltpu.sync_copy(fcw_hbm, fcw_v)
  pltpu.sync_copy(fcb_hbm, fcb_v)
  f = [fcw_v[pl.ds(16 * k, 16)] for k in range(D // 16)]
  fb = fcb_v[...]
  lane15 = lax.iota(jnp.int32, 16) == 15

  def fire_ids(blk, b):
    rb = idrow0 + blk * CB
    pltpu.async_copy(widx_hbm.at[pl.ds(rb, CB)], idx_w.at[b], idsem[b])
    pltpu.async_copy(cidx_hbm.at[pl.ds(rb, CB)], idx_c.at[b], idsem[b])

  def drain_ids(b):
    pltpu.make_async_copy(widx_hbm.at[pl.ds(0, CB)], idx_w.at[b],
                          idsem[b]).wait()
    pltpu.make_async_copy(cidx_hbm.at[pl.ds(0, CB)], idx_c.at[b],
                          idsem[b]).wait()

  def fire_rows(iw_row, ic_row, p):
    pltpu.async_copy(table_hbm.at[iw_row], rows_w.at[p], rowsem[p])
    pltpu.async_copy(table_hbm.at[ic_row], rows_c.at[p], rowsem[p])

  def drain_rows(p):
    pltpu.make_async_copy(table_hbm.at[pl.ds(0, CHUNK)], rows_w.at[p],
                          rowsem[p]).wait()
    pltpu.make_async_copy(table_hbm.at[pl.ds(0, CHUNK)], rows_c.at[p],
                          rowsem[p]).wait()

  def compute_chunk(j, bb, p):
    off = j * CHUNK

    @plsc.parallel_loop(0, CHUNK, 1, unroll=PU)
    def _(pi):
      acc = (rows_w[p, pi, pl.ds(0, 16)]
             * rows_c[p, pi, pl.ds(0, 16)]) * f[0]
      for k in range(1, D // 16):
        acc += (rows_w[p, pi, pl.ds(16 * k, 16)]
                * rows_c[p, pi, pl.ds(16 * k, 16)]) * f[k]
      cs = jnp.cumsum(acc)
      plsc.store_compressed(out_v.at[bb].at[pl.ds(off + pi, 16)], cs,
                            mask=lane15)

  fire_ids(0, 0)
  drain_ids(0)
  fire_rows(idx_w.at[0].at[0], idx_c.at[0].at[0], 0)

  def block_loop(t, _):
    for sb in (0, 1):
      blk = 2 * t + sb
      bb = sb

      @pl.when(blk + 1 < NBLK)
      def _():
        fire_ids(blk + 1, 1 - bb)

      @pl.when(blk >= 2)
      def _():
        pltpu.make_async_copy(out_v.at[bb].at[pl.ds(0, BLKSZ)],
                              out_hbm.at[pl.ds(0, BLKSZ)],
                              outsem[bb]).wait()

      for j in range(CB):
        p = j % 2
        if j < CB - 1:
          fire_rows(idx_w.at[bb].at[j + 1], idx_c.at[bb].at[j + 1], 1 - p)
        else:
          @pl.when(blk + 1 < NBLK)
          def _():
            drain_ids(1 - bb)
            fire_rows(idx_w.at[1 - bb].at[0], idx_c.at[1 - bb].at[0], 1 - p)
        drain_rows(p)
        compute_chunk(j, bb, p)

      def sig_block(s, _):
        v = out_v[bb, pl.ds(s * 16, 16)] + fb
        out_v[bb, pl.ds(s * 16, 16)] = 1.0 / (1.0 + jnp.exp(-v))
        return 0

      lax.fori_loop(0, BLKSZ // 16, sig_block, 0)

      pltpu.async_copy(out_v.at[bb].at[pl.ds(0, BLKSZ)],
                       out_hbm.at[pl.ds(base0 + blk * BLKSZ, BLKSZ)],
                       outsem[bb])
    return 0

  lax.fori_loop(0, NBLK // 2, block_loop, 0)
  for bb in (0, 1):
    pltpu.make_async_copy(out_v.at[bb].at[pl.ds(0, BLKSZ)],
                          out_hbm.at[pl.ds(0, BLKSZ)], outsem[bb]).wait()


@jax.jit
def _run(widx, cidx, table, fcw, fcb16):
  mesh = plsc.VectorSubcoreMesh(
      core_axis_name="c", subcore_axis_name="s",
      num_cores=NC, num_subcores=NS)
  return pl.kernel(
      _body,
      out_type=jax.ShapeDtypeStruct((N,), jnp.float32),
      mesh=mesh,
      compiler_params=pltpu.CompilerParams(
          needs_layout_passes=False, use_tc_tiling_on_sc=False),
      scratch_types=[
          pltpu.VMEM((2, CB, CHUNK), jnp.int32),
          pltpu.VMEM((2, CB, CHUNK), jnp.int32),
          pltpu.VMEM((2, CHUNK, D), jnp.float32),
          pltpu.VMEM((2, CHUNK, D), jnp.float32),
          pltpu.VMEM((2, BLKSZ + LANES), jnp.float32),
          pltpu.VMEM((D,), jnp.float32),
          pltpu.VMEM((LANES,), jnp.float32),
          pltpu.SemaphoreType.DMA,
          pltpu.SemaphoreType.DMA,
          pltpu.SemaphoreType.DMA,
          pltpu.SemaphoreType.DMA,
          pltpu.SemaphoreType.DMA,
          pltpu.SemaphoreType.DMA,
      ],
  )(widx, cidx, table, fcw, fcb16)


def kernel(word_ids, context_ids, table, fc_w, fc_b):
  widx = (word_ids.astype(jnp.int32) * 2).reshape(N // CHUNK, CHUNK)
  cidx = (context_ids.astype(jnp.int32) * 2).reshape(N // CHUNK, CHUNK)
  fcw = fc_w.reshape(D)
  fcb16 = jnp.broadcast_to(fc_b.reshape(1), (LANES,))
  table_rm = _to_row_major(jnp.swapaxes(table, 0, 1))
  out = _run(widx, cidx, table_rm.reshape(2 * 1000000, D), fcw, fcb16)
  return out.reshape(B, L, 1)
